# baseline (speedup 1.0000x reference)
import jax
import jax.numpy as jnp
from jax.experimental import pallas as pl

_EPS = 1e-12
_BLOCK_ROWS = 4000


def _normalize_block(x_ref, o_ref):
    x = x_ref[...]
    d = 64
    half = jnp.ones((d, d), dtype=x.dtype)
    zero = jnp.zeros((d, d), dtype=x.dtype)
    m = jnp.block([[half, zero], [zero, half]])
    s = jax.lax.dot(x * x, m)
    o_ref[...] = x * jax.lax.rsqrt(jnp.maximum(s, _EPS * _EPS))


def kernel(weight):
    n_rows, dim = weight.shape
    packed = weight.reshape(n_rows // 2, 2 * dim)
    grid = packed.shape[0] // _BLOCK_ROWS
    out = pl.pallas_call(
        _normalize_block,
        grid=(grid,),
        in_specs=[pl.BlockSpec((_BLOCK_ROWS, 2 * dim), lambda i: (i, 0))],
        out_specs=pl.BlockSpec((_BLOCK_ROWS, 2 * dim), lambda i: (i, 0)),
        out_shape=jax.ShapeDtypeStruct(packed.shape, packed.dtype),
    )(packed)
    return out.reshape(n_rows, dim)


# speedup vs baseline: 1.3897x; 1.3897x over previous
import jax
import jax.numpy as jnp
from jax.experimental import pallas as pl
from jax.experimental.pallas import tpu as pltpu

_EPS = 1e-12
_K = 8
_CH = 2500
_DIM = 64


def _body(x_hbm, o_hbm, in_buf, out_buf, in_sem, out_sem):
    step = pl.program_id(0)
    nsteps = pl.num_programs(0)
    rows_per_step = _K * _CH

    def in_copy(slot, s):
        base = s * rows_per_step + slot * _CH
        return pltpu.make_async_copy(
            x_hbm.at[pl.ds(base, _CH), :], in_buf.at[slot], in_sem.at[slot])

    def out_copy(slot, s):
        base = s * rows_per_step + slot * _CH
        return pltpu.make_async_copy(
            out_buf.at[slot], o_hbm.at[pl.ds(base, _CH), :], out_sem.at[slot])

    @pl.when(step == 0)
    def _prologue():
        for j in range(_K):
            in_copy(j, 0).start()

    ones = jnp.ones((_DIM, _DIM), dtype=jnp.float32)
    for j in range(_K):
        in_copy(j, step).wait()

        @pl.when(step > 0)
        def _slot_free():
            out_copy(j, step - 1).wait()

        x = in_buf[j]
        s = jax.lax.dot(x * x, ones)
        out_buf[j] = x * jax.lax.rsqrt(jnp.maximum(s, _EPS * _EPS))

        @pl.when(step + 1 < nsteps)
        def _prefetch():
            in_copy(j, step + 1).start()

        out_copy(j, step).start()

    @pl.when(step == nsteps - 1)
    def _epilogue():
        for j in range(_K):
            out_copy(j, step).wait()


def kernel(weight):
    n_rows, dim = weight.shape
    nsteps = n_rows // (_K * _CH)
    return pl.pallas_call(
        _body,
        grid=(nsteps,),
        in_specs=[pl.BlockSpec(memory_space=pltpu.MemorySpace.HBM)],
        out_specs=pl.BlockSpec(memory_space=pltpu.MemorySpace.HBM),
        out_shape=jax.ShapeDtypeStruct((n_rows, dim), weight.dtype),
        scratch_shapes=[
            pltpu.VMEM((_K, _CH, _DIM), jnp.float32),
            pltpu.VMEM((_K, _CH, _DIM), jnp.float32),
            pltpu.SemaphoreType.DMA((_K,)),
            pltpu.SemaphoreType.DMA((_K,)),
        ],
    )(weight)


# speedup vs baseline: 1.3927x; 1.0021x over previous
import jax
import jax.numpy as jnp
from jax.experimental import pallas as pl
from jax.experimental.pallas import tpu as pltpu

_EPS = 1e-12
_K = 10
_CH = 2000
_DIM = 64


def _body(x_hbm, o_hbm, in_buf, out_buf, in_sem, out_sem):
    step = pl.program_id(0)
    nsteps = pl.num_programs(0)
    rows_per_step = _K * _CH

    def in_copy(slot, s):
        base = s * rows_per_step + slot * _CH
        return pltpu.make_async_copy(
            x_hbm.at[pl.ds(base, _CH), :], in_buf.at[slot], in_sem.at[slot])

    def out_copy(slot, s):
        base = s * rows_per_step + slot * _CH
        return pltpu.make_async_copy(
            out_buf.at[slot], o_hbm.at[pl.ds(base, _CH), :], out_sem.at[slot])

    @pl.when(step == 0)
    def _prologue():
        for j in range(_K):
            in_copy(j, 0).start()

    ones = jnp.ones((_DIM, _DIM), dtype=jnp.float32)
    for j in range(_K):
        in_copy(j, step).wait()

        @pl.when(step > 0)
        def _slot_free():
            out_copy(j, step - 1).wait()

        x = in_buf[j]
        s = jax.lax.dot(x * x, ones)
        out_buf[j] = x * jax.lax.rsqrt(jnp.maximum(s, _EPS * _EPS))

        @pl.when(step + 1 < nsteps)
        def _prefetch():
            in_copy(j, step + 1).start()

        out_copy(j, step).start()

    @pl.when(step == nsteps - 1)
    def _epilogue():
        for j in range(_K):
            out_copy(j, step).wait()


def kernel(weight):
    n_rows, dim = weight.shape
    nsteps = n_rows // (_K * _CH)
    return pl.pallas_call(
        _body,
        grid=(nsteps,),
        in_specs=[pl.BlockSpec(memory_space=pltpu.MemorySpace.HBM)],
        out_specs=pl.BlockSpec(memory_space=pltpu.MemorySpace.HBM),
        out_shape=jax.ShapeDtypeStruct((n_rows, dim), weight.dtype),
        scratch_shapes=[
            pltpu.VMEM((_K, _CH, _DIM), jnp.float32),
            pltpu.VMEM((_K, _CH, _DIM), jnp.float32),
            pltpu.SemaphoreType.DMA((_K,)),
            pltpu.SemaphoreType.DMA((_K,)),
        ],
    )(weight)


# speedup vs baseline: 8.7976x; 6.3170x over previous
import jax
import jax.numpy as jnp
from jax.experimental import pallas as pl
from jax.experimental.pallas import tpu as pltpu

_EPS = 1e-12
_K = 12
_CH = 3968
_DIM = 64
_TAIL = 64


def _normalize(x):
    s = jnp.sum(x * x, axis=0, keepdims=True)
    return x * jax.lax.rsqrt(jnp.maximum(s, _EPS * _EPS))


def _body(x_hbm, o_hbm, in_buf, out_buf, tin, tout, in_sem, out_sem, tsem):
    step = pl.program_id(0)
    nsteps = pl.num_programs(0)
    cols_per_step = _K * _CH

    def in_copy(slot, s):
        base = s * cols_per_step + slot * _CH
        return pltpu.make_async_copy(
            x_hbm.at[:, pl.ds(base, _CH)], in_buf.at[slot], in_sem.at[slot])

    def out_copy(slot, s):
        base = s * cols_per_step + slot * _CH
        return pltpu.make_async_copy(
            out_buf.at[slot], o_hbm.at[:, pl.ds(base, _CH)], out_sem.at[slot])

    @pl.when(step == 0)
    def _prologue():
        for j in range(_K):
            in_copy(j, 0).start()

    for j in range(_K):
        in_copy(j, step).wait()

        @pl.when(step > 0)
        def _slot_free():
            out_copy(j, step - 1).wait()

        out_buf[j] = _normalize(in_buf[j])

        @pl.when(step + 1 < nsteps)
        def _prefetch():
            in_copy(j, step + 1).start()

        out_copy(j, step).start()

    @pl.when(step == nsteps - 1)
    def _epilogue():
        base = nsteps * cols_per_step
        pltpu.make_async_copy(
            x_hbm.at[:, pl.ds(base, _TAIL)], tin, tsem).start()
        pltpu.make_async_copy(
            x_hbm.at[:, pl.ds(base, _TAIL)], tin, tsem).wait()
        tout[...] = _normalize(tin[...])
        pltpu.make_async_copy(
            tout, o_hbm.at[:, pl.ds(base, _TAIL)], tsem).start()
        pltpu.make_async_copy(
            tout, o_hbm.at[:, pl.ds(base, _TAIL)], tsem).wait()
        for j in range(_K):
            out_copy(j, step).wait()


def kernel(weight):
    n_rows, dim = weight.shape
    wt = weight.T
    nsteps = (n_rows - _TAIL) // (_K * _CH)
    out_t = pl.pallas_call(
        _body,
        grid=(nsteps,),
        in_specs=[pl.BlockSpec(memory_space=pltpu.MemorySpace.HBM)],
        out_specs=pl.BlockSpec(memory_space=pltpu.MemorySpace.HBM),
        out_shape=jax.ShapeDtypeStruct((dim, n_rows), weight.dtype),
        scratch_shapes=[
            pltpu.VMEM((_K, _DIM, _CH), jnp.float32),
            pltpu.VMEM((_K, _DIM, _CH), jnp.float32),
            pltpu.VMEM((_DIM, _TAIL), jnp.float32),
            pltpu.VMEM((_DIM, _TAIL), jnp.float32),
            pltpu.SemaphoreType.DMA((_K,)),
            pltpu.SemaphoreType.DMA((_K,)),
            pltpu.SemaphoreType.DMA,
        ],
    )(wt)
    return out_t.T
